# baseline (speedup 1.0000x reference)
import functools

import jax
import jax.numpy as jnp
from jax import lax
from jax.experimental import pallas as pl
from jax.experimental.pallas import tpu as pltpu
from jax.experimental.pallas import tpu_sc as plsc

_NC = 2
_NS = 16
_NW = _NC * _NS
_LANES = 16
_CHUNK = 128
_NBUF = 3
_SCHED = (16, 48, 112, 128, 112, 64, 32)


@functools.lru_cache(maxsize=None)
def _make_center_loss(batch, feat):
    b_per_w = batch // _NW
    assert sum(_SCHED) == b_per_w
    n_chunks = len(_SCHED)
    offs = [sum(_SCHED[:i]) for i in range(n_chunks)]
    n_col = feat // _LANES
    mesh = plsc.VectorSubcoreMesh(core_axis_name="c", subcore_axis_name="s")

    @functools.partial(
        pl.kernel,
        mesh=mesh,
        out_type=jax.ShapeDtypeStruct((_NW, _LANES), jnp.float32),
        scratch_types=[
            pltpu.VMEM((b_per_w,), jnp.int32),
            pltpu.VMEM((_NBUF, _CHUNK, feat), jnp.float32),
            pltpu.VMEM((_NBUF, _CHUNK, feat), jnp.float32),
            pltpu.VMEM((_LANES,), jnp.float32),
            pltpu.SemaphoreType.DMA,
            pltpu.SemaphoreType.DMA,
            pltpu.SemaphoreType.DMA,
        ],
    )
    def k(x_hbm, labels_hbm, centers_hbm, out_hbm, idx_v, x_v, rows_v, acc_v,
          sem0, sem1, sem2):
        wid = lax.axis_index("s") * _NC + lax.axis_index("c")
        base = wid * b_per_w
        sems = (sem0, sem1, sem2)

        pltpu.sync_copy(labels_hbm.at[pl.ds(base, b_per_w)], idx_v)

        def start(c):
            slot = c % _NBUF
            sz = _SCHED[c]
            off = offs[c]
            dx = pltpu.async_copy(
                x_hbm.at[pl.ds(base + off, sz)],
                x_v.at[slot, pl.ds(0, sz)], sems[slot])
            dr = pltpu.async_copy(
                centers_hbm.at[idx_v.at[pl.ds(off, sz)]],
                rows_v.at[slot, pl.ds(0, sz)], sems[slot])
            return dx, dr

        zero = jnp.zeros((_LANES,), jnp.float32)
        accs = (zero,) * n_col
        pending = [start(c) for c in range(min(_NBUF - 1, n_chunks))]
        for c in range(n_chunks):
            slot = c % _NBUF
            if c + _NBUF - 1 < n_chunks:
                pending.append(start(c + _NBUF - 1))
            dx, dr = pending.pop(0)
            dx.wait()
            dr.wait()

            def row_body(j, accs, slot=slot):
                new = []
                for t in range(n_col):
                    xv = x_v[slot, j, pl.ds(t * _LANES, _LANES)]
                    rv = rows_v[slot, j, pl.ds(t * _LANES, _LANES)]
                    d = xv - rv
                    new.append(accs[t] + d * d)
                return tuple(new)

            accs = lax.fori_loop(0, _SCHED[c], row_body, accs)

        total = accs[0]
        for t in range(1, n_col):
            total = total + accs[t]
        acc_v[...] = total
        pltpu.sync_copy(acc_v, out_hbm.at[wid])

    return k


def kernel(x, labels, centers):
    partials = _make_center_loss(x.shape[0], x.shape[1])(x, labels, centers)
    return jnp.sum(partials)


# speedup vs baseline: 1.0116x; 1.0116x over previous
import functools

import jax
import jax.numpy as jnp
from jax import lax
from jax.experimental import pallas as pl
from jax.experimental.pallas import tpu as pltpu
from jax.experimental.pallas import tpu_sc as plsc

_NC = 2
_NS = 16
_NW = _NC * _NS
_LANES = 16
_CHUNK = 128
_NBUF = 3
_SCHED = (32, 96, 128, 128, 96, 32)


@functools.lru_cache(maxsize=None)
def _make_center_loss(batch, feat):
    b_per_w = batch // _NW
    assert sum(_SCHED) == b_per_w
    n_chunks = len(_SCHED)
    offs = [sum(_SCHED[:i]) for i in range(n_chunks)]
    n_col = feat // _LANES
    mesh = plsc.VectorSubcoreMesh(core_axis_name="c", subcore_axis_name="s")

    @functools.partial(
        pl.kernel,
        mesh=mesh,
        out_type=jax.ShapeDtypeStruct((_NW, _LANES), jnp.float32),
        scratch_types=[
            pltpu.VMEM((b_per_w,), jnp.int32),
            pltpu.VMEM((_NBUF, _CHUNK, feat), jnp.float32),
            pltpu.VMEM((_NBUF, _CHUNK, feat), jnp.float32),
            pltpu.VMEM((_LANES,), jnp.float32),
            pltpu.SemaphoreType.DMA,
            pltpu.SemaphoreType.DMA,
            pltpu.SemaphoreType.DMA,
        ],
    )
    def k(x_hbm, labels_hbm, centers_hbm, out_hbm, idx_v, x_v, rows_v, acc_v,
          sem0, sem1, sem2):
        wid = lax.axis_index("s") * _NC + lax.axis_index("c")
        base = wid * b_per_w
        sems = (sem0, sem1, sem2)

        pltpu.sync_copy(labels_hbm.at[pl.ds(base, b_per_w)], idx_v)

        def start(c):
            slot = c % _NBUF
            sz = _SCHED[c]
            off = offs[c]
            dx = pltpu.async_copy(
                x_hbm.at[pl.ds(base + off, sz)],
                x_v.at[slot, pl.ds(0, sz)], sems[slot])
            dr = pltpu.async_copy(
                centers_hbm.at[idx_v.at[pl.ds(off, sz)]],
                rows_v.at[slot, pl.ds(0, sz)], sems[slot])
            return dx, dr

        zero = jnp.zeros((_LANES,), jnp.float32)
        accs = (zero,) * n_col
        pending = [start(c) for c in range(min(_NBUF - 1, n_chunks))]
        for c in range(n_chunks):
            slot = c % _NBUF
            if c + _NBUF - 1 < n_chunks:
                pending.append(start(c + _NBUF - 1))
            dx, dr = pending.pop(0)
            dx.wait()
            dr.wait()

            def row_body(j, accs, slot=slot):
                new = []
                for t in range(n_col):
                    xv = x_v[slot, j, pl.ds(t * _LANES, _LANES)]
                    rv = rows_v[slot, j, pl.ds(t * _LANES, _LANES)]
                    d = xv - rv
                    new.append(accs[t] + d * d)
                return tuple(new)

            accs = lax.fori_loop(0, _SCHED[c], row_body, accs)

        total = accs[0]
        for t in range(1, n_col):
            total = total + accs[t]
        acc_v[...] = total
        pltpu.sync_copy(acc_v, out_hbm.at[wid])

    return k


def kernel(x, labels, centers):
    partials = _make_center_loss(x.shape[0], x.shape[1])(x, labels, centers)
    return jnp.sum(partials)
